# Initial kernel scaffold; baseline (speedup 1.0000x reference)
import jax
import jax.numpy as jnp
from jax.experimental import pallas as pl


def kernel(x, edge_index, edge_weight, W0, W1):
    raise NotImplementedError("write your pallas kernel here")


# speedup vs baseline: 3.5042x; 3.5042x over previous
import functools

import jax
import jax.numpy as jnp
from jax import lax
from jax.experimental import pallas as pl
from jax.experimental.pallas import tpu as pltpu
from jax.experimental.pallas import tpu_sc as plsc

NC = 2
NS = 16
NW = NC * NS
BINS = NW
CHUNK = 128
L = 16


def _rsqrt16(d):
    i = lax.bitcast_convert_type(d, jnp.int32)
    i = jnp.int32(0x5F3759DF) - (i >> 1)
    y = lax.bitcast_convert_type(i, jnp.float32)
    for _ in range(3):
        y = y * (1.5 - 0.5 * d * y * y)
    return y


def _make_norm_kernel(n_pad, kp, kd):
    npt = n_pad // NS
    mesh = plsc.VectorSubcoreMesh(core_axis_name="c", subcore_axis_name="s", num_cores=NC, num_subcores=NS)

    def body(row3, col3, ew3, norm3, counts, degacc, zbuf, rowA, colA, ewA,
             deg_v, normF, cnt16, dsem):
        s = lax.axis_index("s")
        c = lax.axis_index("c")
        wid = s * 2 + c
        iota = lax.iota(jnp.int32, L)

        def zb(i, _):
            zbuf[pl.ds(i * L, L)] = jnp.zeros((L,), jnp.float32)
            return _
        lax.fori_loop(0, npt // L, zb, None)
        pltpu.sync_copy(zbuf, degacc.at[pl.ds(s * npt, npt)])
        plsc.subcore_barrier()

        for wo in range(2):
            w = s * 2 + wo
            pltpu.sync_copy(col3.at[w], colA)
            pltpu.sync_copy(ew3.at[w], ewA)

            def dg(k, _):
                pltpu.async_copy(ewA.at[k], degacc.at[colA.at[k]], dsem,
                                 add=True)
                return _
            lax.fori_loop(0, kp, dg, None)

            def dgw(k, _):
                pltpu.make_async_copy(ewA.at[0], degacc.at[colA.at[0]],
                                      dsem).wait()
                return _
            lax.fori_loop(0, kp, dgw, None)
        plsc.subcore_barrier()

        pltpu.sync_copy(degacc, deg_v)

        def rs(i, _):
            sl = pl.ds(i * L, L)
            deg_v[sl] = _rsqrt16(deg_v[sl])
            return _
        lax.fori_loop(0, n_pad // L, rs, None)

        pltpu.sync_copy(row3.at[wid], rowA)
        pltpu.sync_copy(col3.at[wid], colA)
        pltpu.sync_copy(ew3.at[wid], ewA)

        def nk(k, _):
            for b in range(CHUNK // L):
                sl = pl.ds(b * L, L)
                r16 = rowA[k, sl]
                c16 = colA[k, sl]
                e16 = ewA[k, sl]
                dr = plsc.load_gather(deg_v, [r16])
                dc = plsc.load_gather(deg_v, [c16])
                normF[k, sl] = dr * e16 * dc
            return _
        lax.fori_loop(0, kd, nk, None)
        pltpu.sync_copy(normF, norm3.at[wid])

        def zc(i, _):
            cnt16[pl.ds(i * L, L)] = jnp.zeros((L,), jnp.int32)
            return _
        lax.fori_loop(0, (BINS * L) // L, zc, None)

        def cc(k, _):
            for b in range(CHUNK // L):
                c16 = colA[k, pl.ds(b * L, L)]
                idx = (c16 & (BINS - 1)) * L + iota
                cv = plsc.load_gather(cnt16, [idx])
                plsc.store_scatter(cnt16, [idx], cv + 1)
            return _
        lax.fori_loop(0, kp, cc, None)
        pltpu.sync_copy(cnt16, counts.at[wid])

    return pl.kernel(
        body,
        out_type=(jax.ShapeDtypeStruct((NW, kd, CHUNK), jnp.float32),
                  jax.ShapeDtypeStruct((NW, BINS * L), jnp.int32)),
        mesh=mesh,
        compiler_params=pltpu.CompilerParams(needs_layout_passes=False),
        scratch_types=[
            pltpu.VMEM_SHARED((n_pad,), jnp.float32),
            pltpu.VMEM((npt,), jnp.float32),
            pltpu.VMEM((kd, CHUNK), jnp.int32),
            pltpu.VMEM((kd, CHUNK), jnp.int32),
            pltpu.VMEM((kd, CHUNK), jnp.float32),
            pltpu.VMEM((n_pad,), jnp.float32),
            pltpu.VMEM((kd, CHUNK), jnp.float32),
            pltpu.VMEM((BINS * L,), jnp.int32),
            pltpu.SemaphoreType.DMA,
        ],
    )


def _make_route_kernel(kp, kd, en2, en2t):
    mesh = plsc.VectorSubcoreMesh(core_axis_name="c", subcore_axis_name="s", num_cores=NC, num_subcores=NS)

    def body(row3, col3, norm3, counts, rowR, lrR, normR, cnts, Iv, Otile,
             crun, rowS, colS, normS, posS, sbin, rsem):
        s = lax.axis_index("s")
        c = lax.axis_index("c")
        wid = s * 2 + c
        iota = lax.iota(jnp.int32, L)

        pltpu.sync_copy(counts, cnts)
        pltpu.sync_copy(row3.at[wid], rowS)
        pltpu.sync_copy(col3.at[wid], colS)
        pltpu.sync_copy(norm3.at[wid], normS)

        def ob(b, carry):
            sbin[2 * BINS + b] = carry

            def ot(t, carry2):
                ct = cnts[t, pl.ds(b * L, L)]
                incl = plsc.cumsum(ct)
                Iv[b, pl.ds(t * L, L)] = (incl - ct) + carry2
                return carry2 + jnp.sum(ct)
            carry_end = lax.fori_loop(0, NW, ot, carry)
            sbin[b] = carry_end - carry
            return carry_end
        lax.fori_loop(0, BINS, ob, jnp.int32(0))

        def pb(b, run):
            sbin[BINS + b] = run
            return run + (((sbin[b] + CHUNK - 1) >> 7) << 7)
        lax.fori_loop(0, BINS, pb, jnp.int32(0))

        def otl(b, _):
            v = Iv[b, pl.ds(wid * L, L)]
            adj = jnp.full((L,), sbin[BINS + b] - sbin[2 * BINS + b],
                           jnp.int32)
            Otile[pl.ds(b * L, L)] = v + adj
            return _
        lax.fori_loop(0, BINS, otl, None)

        def zc(i, _):
            crun[pl.ds(i * L, L)] = jnp.zeros((L,), jnp.int32)
            return _
        lax.fori_loop(0, BINS, zc, None)

        def pk(k, _):
            for g in range(CHUNK // L):
                sl = pl.ds(g * L, L)
                c16 = colS[k, sl]
                idx = (c16 & (BINS - 1)) * L + iota
                base = plsc.load_gather(Otile, [idx])
                r = plsc.load_gather(crun, [idx])
                plsc.store_scatter(crun, [idx], r + 1)
                posS[k, sl] = base + r
                colS[k, sl] = c16 >> 5
            return _
        lax.fori_loop(0, kp, pk, None)

        bs = sbin[BINS + wid]
        tw = sbin[wid]
        gap = ((((tw + CHUNK - 1) >> 7) << 7)) - tw
        for g in range(CHUNK // L):
            sl = pl.ds(g * L, L)
            i16 = iota + (g * L)
            fill = jnp.full((L,), bs + tw, jnp.int32) + i16
            trash = jnp.full((L,), en2, jnp.int32) + i16
            posS[kp, sl] = jnp.where(i16 < gap, fill, trash)
            rowS[kp, sl] = jnp.zeros((L,), jnp.int32)
            colS[kp, sl] = jnp.zeros((L,), jnp.int32)
            normS[kp, sl] = jnp.zeros((L,), jnp.float32)

        def fire(k):
            pltpu.async_copy(rowS.at[k], rowR.at[posS.at[k]], rsem)
            pltpu.async_copy(colS.at[k], lrR.at[posS.at[k]], rsem)
            pltpu.async_copy(normS.at[k], normR.at[posS.at[k]], rsem)

        def drain1():
            pltpu.make_async_copy(rowS.at[0], rowR.at[posS.at[0]], rsem).wait()
            pltpu.make_async_copy(colS.at[0], lrR.at[posS.at[0]], rsem).wait()
            pltpu.make_async_copy(normS.at[0], normR.at[posS.at[0]],
                                  rsem).wait()

        LAG = 8

        def so(k, _):
            fire(k)

            @pl.when(k >= LAG)
            def _w():
                drain1()
            return _
        lax.fori_loop(0, kp + 1, so, None)

        def sd(k, _):
            drain1()
            return _
        lax.fori_loop(0, min(LAG, kp + 1), sd, None)

    return pl.kernel(
        body,
        out_type=(jax.ShapeDtypeStruct((en2t,), jnp.int32),
                  jax.ShapeDtypeStruct((en2t,), jnp.int32),
                  jax.ShapeDtypeStruct((en2t,), jnp.float32)),
        mesh=mesh,
        compiler_params=pltpu.CompilerParams(needs_layout_passes=False),
        scratch_types=[
            pltpu.VMEM((NW, BINS * L), jnp.int32),
            pltpu.VMEM((BINS, NW * L), jnp.int32),
            pltpu.VMEM((BINS * L,), jnp.int32),
            pltpu.VMEM((BINS * L,), jnp.int32),
            pltpu.VMEM((kd, CHUNK), jnp.int32),
            pltpu.VMEM((kd, CHUNK), jnp.int32),
            pltpu.VMEM((kd, CHUNK), jnp.float32),
            pltpu.VMEM((kd, CHUNK), jnp.int32),
            pltpu.SMEM((3 * BINS,), jnp.int32),
            pltpu.SemaphoreType.DMA,
        ],
    )


def _make_agg_kernel(n_pad):
    npb = n_pad // BINS
    mesh = plsc.VectorSubcoreMesh(core_axis_name="c", subcore_axis_name="s", num_cores=NC, num_subcores=NS)

    def body(xw, rowR, lrR, normR, counts, out, accL, cnts, rowC, lrC, normC,
             buf0, buf1, fsem0, fsem1, isem0, isem1):
        s = lax.axis_index("s")
        c = lax.axis_index("c")
        wid = s * 2 + c
        iota = lax.iota(jnp.int32, L)

        def za(r, _):
            for f in range(CHUNK // L):
                accL[r, pl.ds(f * L, L)] = jnp.zeros((L,), jnp.float32)
            return _
        lax.fori_loop(0, npb, za, None)

        pltpu.sync_copy(counts, cnts)

        def bb(b, car):
            bs_, tw_ = car

            def st(t, v):
                return v + cnts[t, pl.ds(b * L, L)]
            s16 = lax.fori_loop(0, NW, st, jnp.zeros((L,), jnp.int32))
            tb = jnp.sum(s16)
            pb = ((tb + CHUNK - 1) >> 7) << 7
            bs_ = bs_ + jnp.where(b < wid, pb, 0)
            tw_ = jnp.where(b == wid, tb, tw_)
            return (bs_, tw_)
        bs, tw = lax.fori_loop(0, BINS, bb,
                               (jnp.int32(0), jnp.int32(0)))
        cbase = bs >> 7
        nck = (tw + CHUNK - 1) >> 7

        def stage(kk, cr, isem):
            for src, dst in ((rowR, rowC), (lrR, lrC), (normR, normC)):
                pltpu.async_copy(src.at[cbase + kk], dst.at[cr], isem)

        def stage_wait(cr, isem):
            for src, dst in ((rowR, rowC), (lrR, lrC), (normR, normC)):
                pltpu.make_async_copy(src.at[cbase], dst.at[cr], isem).wait()

        iotaL = lax.iota(jnp.int32, L)

        def accumulate(buf, slot):
            slotv = jnp.full((L,), slot, jnp.int32)

            def ag(g, _):
                for l in range(L):
                    e = g * L + l
                    ev = jnp.full((L,), e, jnp.int32)
                    nb = plsc.load_gather(normC, [slotv, ev])
                    lrb = plsc.load_gather(lrC, [slotv, ev])
                    for f in range(CHUNK // L):
                        sl = pl.ds(f * L, L)
                        plsc.addupdate_scatter(accL, [lrb, (f * L) + iotaL],
                                               buf[e, sl] * nb)
                return _
            lax.fori_loop(0, CHUNK // L, ag, None)

        @pl.when(nck >= 1)
        def _p0():
            stage(0, 0, isem0)

        @pl.when(nck >= 2)
        def _p1():
            stage(1, 1, isem1)

        @pl.when(nck >= 1)
        def _p2():
            stage_wait(0, isem0)
            pltpu.async_copy(xw.at[rowC.at[0]], buf0, fsem0)

        def part(k, slot, o, buf_s, buf_o, fsem_s, fsem_o, isem_s, isem_o):
            pltpu.make_async_copy(xw.at[rowC.at[slot]], buf_s, fsem_s).wait()

            @pl.when(k + 1 < nck)
            def _n():
                stage_wait(o, isem_o)
                pltpu.async_copy(xw.at[rowC.at[o]], buf_o, fsem_o)
            accumulate(buf_s, slot)

            @pl.when(k + 2 < nck)
            def _s():
                stage(k + 2, slot, isem_s)

        def step(k, _):
            @pl.when((k & 1) == 0)
            def _e():
                part(k, 0, 1, buf0, buf1, fsem0, fsem1, isem0, isem1)

            @pl.when((k & 1) == 1)
            def _o():
                part(k, 1, 0, buf1, buf0, fsem1, fsem0, isem1, isem0)
            return _
        lax.fori_loop(0, nck, step, None)

        pltpu.sync_copy(accL, out.at[c, s])

    return pl.kernel(
        body,
        out_type=jax.ShapeDtypeStruct((NC, NS, npb, CHUNK), jnp.float32),
        mesh=mesh,
        compiler_params=pltpu.CompilerParams(needs_layout_passes=False),
        scratch_types=[
            pltpu.VMEM((npb, CHUNK), jnp.float32),
            pltpu.VMEM((NW, BINS * L), jnp.int32),
            pltpu.VMEM((2, CHUNK), jnp.int32),
            pltpu.VMEM((2, CHUNK), jnp.int32),
            pltpu.VMEM((2, CHUNK), jnp.float32),
            pltpu.VMEM((CHUNK, CHUNK), jnp.float32),
            pltpu.VMEM((CHUNK, CHUNK), jnp.float32),
            pltpu.SemaphoreType.DMA,
            pltpu.SemaphoreType.DMA,
            pltpu.SemaphoreType.DMA,
            pltpu.SemaphoreType.DMA,
        ],
    )


def _mm_body(x_ref, w_ref, o_ref):
    o_ref[...] = jnp.dot(x_ref[...], w_ref[...],
                         preferred_element_type=jnp.float32)


def _cmm_body(p_ref, w_ref, o_ref):
    h = jnp.maximum(p_ref[...], 0.0)
    o_ref[...] = jnp.dot(h, w_ref[...], preferred_element_type=jnp.float32)


def _fin_body(p_ref, o_ref):
    o_ref[...] = jnp.maximum(p_ref[...], 0.0)


def kernel(x, edge_index, edge_weight, W0, W1):
    n, d_in = x.shape
    e = edge_weight.shape[0]
    n_pad = ((n + NS * CHUNK - 1) // (NS * CHUNK)) * (NS * CHUNK)
    en = e + n
    per = NW * CHUNK
    kp = ((en + per - 1) // per + 1) // 2 * 2
    kd = ((kp + 2 + 7) // 8) * 8
    ep = kp * per
    en2 = ep + NW * CHUNK
    en2t = en2 + 4 * CHUNK

    row = jnp.concatenate([edge_index[0].astype(jnp.int32),
                           jnp.arange(n, dtype=jnp.int32)])
    col = jnp.concatenate([edge_index[1].astype(jnp.int32),
                           jnp.arange(n, dtype=jnp.int32)])
    ew = jnp.concatenate([edge_weight, jnp.ones((n,), jnp.float32)])
    padv = jnp.arange(ep - en, dtype=jnp.int32) & (BINS - 1)
    pad3 = ((0, 0), (0, kd - kp), (0, 0))
    row3 = jnp.pad(jnp.concatenate([row, jnp.zeros((ep - en,), jnp.int32)])
                   .reshape(NW, kp, CHUNK), pad3)
    col3 = jnp.pad(jnp.concatenate([col, padv]).reshape(NW, kp, CHUNK), pad3)
    ew3 = jnp.pad(jnp.concatenate([ew, jnp.zeros((ep - en,), jnp.float32)])
                  .reshape(NW, kp, CHUNK), pad3)
    x_pad = jnp.pad(x, ((0, n_pad - n), (0, 0)))

    norm3, counts = _make_norm_kernel(n_pad, kp, kd)(row3, col3, ew3)
    if False:
        cols_all = col3[:, :kp].reshape(-1)
        rows_all = row3[:, :kp].reshape(-1)
        norm_all = norm3[:, :kp].reshape(-1)
        bins = cols_all & (BINS - 1)
        T = jnp.zeros((BINS,), jnp.int32).at[bins].add(1)
        Tp = ((T + CHUNK - 1) // CHUNK) * CHUNK
        binstart = jnp.concatenate(
            [jnp.zeros((1,), jnp.int32), jnp.cumsum(Tp)[:-1]])
        ustart = jnp.concatenate(
            [jnp.zeros((1,), jnp.int32), jnp.cumsum(T)[:-1]])
        order = jnp.argsort(bins, stable=True)
        sb = bins[order]
        dst = binstart[sb] + (jnp.arange(sb.shape[0], dtype=jnp.int32)
                              - ustart[sb])
        rowR = jnp.zeros((en2t,), jnp.int32).at[dst].set(rows_all[order])
        lrR = jnp.zeros((en2t,), jnp.int32).at[dst].set(
            (cols_all >> 5)[order])
        normR = jnp.zeros((en2t,), jnp.float32).at[dst].set(norm_all[order])
        counts = jnp.zeros((NW, BINS * L), jnp.int32).at[
            0, jnp.arange(BINS) * L].set(T)
    else:
        rowR, lrR, normR = _make_route_kernel(kp, kd, en2, en2t)(
            row3, col3, norm3, counts)

    d_h0 = W0.shape[1]
    grid = (n_pad // 1024,)
    y0 = pl.pallas_call(
        _mm_body,
        grid=grid,
        in_specs=[pl.BlockSpec((1024, d_in), lambda i: (i, 0)),
                  pl.BlockSpec((d_in, d_h0), lambda i: (0, 0))],
        out_specs=pl.BlockSpec((1024, d_h0), lambda i: (i, 0)),
        out_shape=jax.ShapeDtypeStruct((n_pad, d_h0), jnp.float32),
    )(x_pad, W0)

    rowR = rowR.reshape(-1, CHUNK)
    lrR = lrR.reshape(-1, CHUNK)
    normR = normR.reshape(-1, CHUNK)
    agg = _make_agg_kernel(n_pad)

    def unperm(p):
        return jnp.transpose(p, (2, 1, 0, 3)).reshape(n_pad, CHUNK)

    p0 = unperm(agg(y0, rowR, lrR, normR, counts))

    d_h1 = W1.shape[1]
    y1 = pl.pallas_call(
        _cmm_body,
        grid=grid,
        in_specs=[pl.BlockSpec((1024, d_h0), lambda i: (i, 0)),
                  pl.BlockSpec((d_h0, d_h1), lambda i: (0, 0))],
        out_specs=pl.BlockSpec((1024, d_h1), lambda i: (i, 0)),
        out_shape=jax.ShapeDtypeStruct((n_pad, d_h1), jnp.float32),
    )(p0, W1)

    p1 = unperm(agg(y1, rowR, lrR, normR, counts))

    out = pl.pallas_call(
        _fin_body,
        grid=grid,
        in_specs=[pl.BlockSpec((1024, d_h1), lambda i: (i, 0))],
        out_specs=pl.BlockSpec((1024, d_h1), lambda i: (i, 0)),
        out_shape=jax.ShapeDtypeStruct((n_pad, d_h1), jnp.float32),
    )(p1)
    return out[:n]


# speedup vs baseline: 8.8152x; 2.5156x over previous
import functools

import jax
import jax.numpy as jnp
from jax import lax
from jax.experimental import pallas as pl
from jax.experimental.pallas import tpu as pltpu
from jax.experimental.pallas import tpu_sc as plsc

NC = 2
NS = 16
NW = NC * NS
CHUNK = 128
L = 16


def _rsqrt16(d):
    i = lax.bitcast_convert_type(d, jnp.int32)
    i = jnp.int32(0x5F3759DF) - (i >> 1)
    y = lax.bitcast_convert_type(i, jnp.float32)
    for _ in range(3):
        y = y * (1.5 - 0.5 * d * y * y)
    return y


def _make_norm_kernel(n_pad, kp, kd):
    npt = n_pad // NS
    mesh = plsc.VectorSubcoreMesh(core_axis_name="c", subcore_axis_name="s")

    def body(row3, col3, ew3, norm3, degacc, zbuf, rowA, colA, ewA, deg_v,
             normF, dsem):
        s = lax.axis_index("s")
        c = lax.axis_index("c")
        wid = s * 2 + c

        def zb(i, _):
            zbuf[pl.ds(i * L, L)] = jnp.zeros((L,), jnp.float32)
            return _
        lax.fori_loop(0, npt // L, zb, None)
        pltpu.sync_copy(zbuf, degacc.at[pl.ds(s * npt, npt)])
        plsc.subcore_barrier()

        for wo in range(2):
            w = s * 2 + wo
            pltpu.sync_copy(col3.at[w], colA)
            pltpu.sync_copy(ew3.at[w], ewA)

            def dg(k, _):
                pltpu.async_copy(ewA.at[k], degacc.at[colA.at[k]], dsem,
                                 add=True)
                return _
            lax.fori_loop(0, kp, dg, None)

            def dgw(k, _):
                pltpu.make_async_copy(ewA.at[0], degacc.at[colA.at[0]],
                                      dsem).wait()
                return _
            lax.fori_loop(0, kp, dgw, None)
        plsc.subcore_barrier()

        pltpu.sync_copy(degacc, deg_v)

        def rs(i, _):
            sl = pl.ds(i * L, L)
            deg_v[sl] = _rsqrt16(deg_v[sl])
            return _
        lax.fori_loop(0, n_pad // L, rs, None)

        pltpu.sync_copy(row3.at[wid], rowA)
        pltpu.sync_copy(col3.at[wid], colA)
        pltpu.sync_copy(ew3.at[wid], ewA)

        def nk(k, _):
            for b in range(CHUNK // L):
                sl = pl.ds(b * L, L)
                r16 = rowA[k, sl]
                c16 = colA[k, sl]
                e16 = ewA[k, sl]
                dr = plsc.load_gather(deg_v, [r16])
                dc = plsc.load_gather(deg_v, [c16])
                normF[k, sl] = dr * e16 * dc
            return _
        lax.fori_loop(0, kd, nk, None)
        pltpu.sync_copy(normF, norm3.at[wid])

    return pl.kernel(
        body,
        out_type=jax.ShapeDtypeStruct((NW, kd, CHUNK), jnp.float32),
        mesh=mesh,
        compiler_params=pltpu.CompilerParams(needs_layout_passes=False),
        scratch_types=[
            pltpu.VMEM_SHARED((n_pad,), jnp.float32),
            pltpu.VMEM((npt,), jnp.float32),
            pltpu.VMEM((kd, CHUNK), jnp.int32),
            pltpu.VMEM((kd, CHUNK), jnp.int32),
            pltpu.VMEM((kd, CHUNK), jnp.float32),
            pltpu.VMEM((n_pad,), jnp.float32),
            pltpu.VMEM((kd, CHUNK), jnp.float32),
            pltpu.SemaphoreType.DMA,
        ],
    )


def _make_agg_kernel(n_pad, kp, kd):
    npt = n_pad // NS
    mesh = plsc.VectorSubcoreMesh(core_axis_name="c", subcore_axis_name="s")

    def scale(buf, normC, slot):
        def sc_grp(g, _):
            n16 = normC[slot, pl.ds(g * L, L)]
            for l in range(L):
                r = g * L + l
                nb = jnp.full((L,), n16[l], jnp.float32)
                for f in range(CHUNK // L):
                    sl = pl.ds(f * L, L)
                    buf[r, sl] = buf[r, sl] * nb
            return _
        lax.fori_loop(0, CHUNK // L, sc_grp, None)

    def body(xw, row3, col3, norm3, out, acc, rowC, colC, normC, buf0, buf1,
             fsem0, fsem1, isem0, isem1, ssem0, ssem1):
        s = lax.axis_index("s")
        c = lax.axis_index("c")
        wid = s * 2 + c

        def zb(r, _):
            for f in range(CHUNK // L):
                buf0[r, pl.ds(f * L, L)] = jnp.zeros((L,), jnp.float32)
                buf1[r, pl.ds(f * L, L)] = jnp.zeros((L,), jnp.float32)
            return _
        lax.fori_loop(0, CHUNK, zb, None)
        for j in range(npt // CHUNK):
            pltpu.sync_copy(buf0, acc.at[pl.ds(s * npt + j * CHUNK, CHUNK)])
        plsc.subcore_barrier()

        def stage(kk, slot, isem):
            pltpu.async_copy(row3.at[wid, kk], rowC.at[slot], isem)
            pltpu.async_copy(col3.at[wid, kk], colC.at[slot], isem)
            pltpu.async_copy(norm3.at[wid, kk], normC.at[slot], isem)

        def stage_wait(slot, isem):
            for ref, cr in ((row3, rowC), (col3, colC), (norm3, normC)):
                pltpu.make_async_copy(ref.at[wid, 0], cr.at[slot], isem).wait()

        stage(0, 0, isem0)
        stage(1, 1, isem1)
        stage_wait(0, isem0)
        pltpu.async_copy(buf1, acc.at[colC.at[0]], ssem1, add=True)
        pltpu.async_copy(xw.at[rowC.at[0]], buf0, fsem0)

        def part(k, slot, o, buf_s, buf_o, fsem_s, fsem_o, isem_s, isem_o,
                 ssem_s, ssem_o):
            pltpu.make_async_copy(xw.at[rowC.at[slot]], buf_s, fsem_s).wait()
            pltpu.make_async_copy(buf_o, acc.at[colC.at[o]], ssem_o).wait()
            stage_wait(o, isem_o)
            pltpu.async_copy(xw.at[rowC.at[o]], buf_o, fsem_o)
            scale(buf_s, normC, slot)
            pltpu.async_copy(buf_s, acc.at[colC.at[slot]], ssem_s, add=True)
            stage(k + 2, slot, isem_s)

        def step(i, _):
            part(2 * i, 0, 1, buf0, buf1, fsem0, fsem1, isem0, isem1,
                 ssem0, ssem1)
            part(2 * i + 1, 1, 0, buf1, buf0, fsem1, fsem0, isem1, isem0,
                 ssem1, ssem0)
            return _
        lax.fori_loop(0, kp // 2, step, None)
        pltpu.make_async_copy(xw.at[rowC.at[0]], buf0, fsem0).wait()
        stage_wait(1, isem1)
        pltpu.make_async_copy(buf1, acc.at[colC.at[1]], ssem1).wait()
        plsc.subcore_barrier()

        for j in range(npt // CHUNK):
            sl = pl.ds(s * npt + j * CHUNK, CHUNK)
            pltpu.sync_copy(acc.at[sl], out.at[c, sl])

    return pl.kernel(
        body,
        out_type=jax.ShapeDtypeStruct((NC, n_pad, CHUNK), jnp.float32),
        mesh=mesh,
        compiler_params=pltpu.CompilerParams(needs_layout_passes=False),
        scratch_types=[
            pltpu.VMEM_SHARED((n_pad, CHUNK), jnp.float32),
            pltpu.VMEM((2, CHUNK), jnp.int32),
            pltpu.VMEM((2, CHUNK), jnp.int32),
            pltpu.VMEM((2, CHUNK), jnp.float32),
            pltpu.VMEM((CHUNK, CHUNK), jnp.float32),
            pltpu.VMEM((CHUNK, CHUNK), jnp.float32),
            pltpu.SemaphoreType.DMA,
            pltpu.SemaphoreType.DMA,
            pltpu.SemaphoreType.DMA,
            pltpu.SemaphoreType.DMA,
            pltpu.SemaphoreType.DMA,
            pltpu.SemaphoreType.DMA,
        ],
    )


def _mm_body(x_ref, w_ref, o_ref):
    o_ref[...] = jnp.dot(x_ref[...], w_ref[...],
                         preferred_element_type=jnp.float32)


def _cmm_body(p_ref, w_ref, o_ref):
    h = jnp.maximum(p_ref[0] + p_ref[1], 0.0)
    o_ref[...] = jnp.dot(h, w_ref[...], preferred_element_type=jnp.float32)


def _fin_body(p_ref, o_ref):
    o_ref[...] = jnp.maximum(p_ref[0] + p_ref[1], 0.0)


def kernel(x, edge_index, edge_weight, W0, W1):
    n, d_in = x.shape
    e = edge_weight.shape[0]
    n_pad = ((n + NS * CHUNK - 1) // (NS * CHUNK)) * (NS * CHUNK)
    en = e + n
    per = NW * CHUNK
    kp = ((en + per - 1) // per + 1) // 2 * 2
    kd = ((kp + 2 + 7) // 8) * 8
    ep = kp * per

    row = jnp.concatenate([edge_index[0].astype(jnp.int32),
                           jnp.arange(n, dtype=jnp.int32)])
    col = jnp.concatenate([edge_index[1].astype(jnp.int32),
                           jnp.arange(n, dtype=jnp.int32)])
    ew = jnp.concatenate([edge_weight, jnp.ones((n,), jnp.float32)])
    pad3 = ((0, 0), (0, kd - kp), (0, 0))
    row3 = jnp.pad(jnp.pad(row, (0, ep - en)).reshape(NW, kp, CHUNK), pad3)
    col3 = jnp.pad(jnp.pad(col, (0, ep - en)).reshape(NW, kp, CHUNK), pad3)
    ew3 = jnp.pad(jnp.pad(ew, (0, ep - en)).reshape(NW, kp, CHUNK), pad3)
    x_pad = jnp.pad(x, ((0, n_pad - n), (0, 0)))

    norm3 = _make_norm_kernel(n_pad, kp, kd)(row3, col3, ew3)

    d_h0 = W0.shape[1]
    grid = (n_pad // 1024,)
    y0 = pl.pallas_call(
        _mm_body,
        grid=grid,
        in_specs=[pl.BlockSpec((1024, d_in), lambda i: (i, 0)),
                  pl.BlockSpec((d_in, d_h0), lambda i: (0, 0))],
        out_specs=pl.BlockSpec((1024, d_h0), lambda i: (i, 0)),
        out_shape=jax.ShapeDtypeStruct((n_pad, d_h0), jnp.float32),
    )(x_pad, W0)

    agg = _make_agg_kernel(n_pad, kp, kd)
    p0 = agg(y0, row3, col3, norm3)

    d_h1 = W1.shape[1]
    y1 = pl.pallas_call(
        _cmm_body,
        grid=grid,
        in_specs=[pl.BlockSpec((NC, 1024, d_h0), lambda i: (0, i, 0)),
                  pl.BlockSpec((d_h0, d_h1), lambda i: (0, 0))],
        out_specs=pl.BlockSpec((1024, d_h1), lambda i: (i, 0)),
        out_shape=jax.ShapeDtypeStruct((n_pad, d_h1), jnp.float32),
    )(p0, W1)

    p1 = agg(y1, row3, col3, norm3)

    out = pl.pallas_call(
        _fin_body,
        grid=grid,
        in_specs=[pl.BlockSpec((NC, 1024, d_h1), lambda i: (0, i, 0))],
        out_specs=pl.BlockSpec((1024, d_h1), lambda i: (i, 0)),
        out_shape=jax.ShapeDtypeStruct((n_pad, d_h1), jnp.float32),
    )(p1)
    return out[:n]
